# baseline (speedup 1.0000x reference)
import jax
import jax.numpy as jnp
from jax.experimental import pallas as pl

_ROWS = 37632
_COLS = 1024
_BLOCK_ROWS = 768


def _discretize_body(w_ref, o_ref):
    x = jnp.clip(w_ref[...], 0.0, 1.0)
    o_ref[...] = jnp.round(x * 255.0) / 255.0


def kernel(watermark_samples, response):
    flat = watermark_samples.reshape(_ROWS, _COLS)
    out = pl.pallas_call(
        _discretize_body,
        grid=(_ROWS // _BLOCK_ROWS,),
        in_specs=[pl.BlockSpec((_BLOCK_ROWS, _COLS), lambda i: (i, 0))],
        out_specs=pl.BlockSpec((_BLOCK_ROWS, _COLS), lambda i: (i, 0)),
        out_shape=jax.ShapeDtypeStruct((_ROWS, _COLS), jnp.float32),
    )(flat)
    return (out.reshape(watermark_samples.shape), response)


# speedup vs baseline: 2.0452x; 2.0452x over previous
import jax
import jax.numpy as jnp
from jax.experimental import pallas as pl

_ROWS = 172032
_COLS = 224
_BLOCK_ROWS = 4096


def _discretize_body(w_ref, o_ref):
    x = jnp.clip(w_ref[...], 0.0, 1.0)
    o_ref[...] = jnp.round(x * 255.0) / 255.0


def kernel(watermark_samples, response):
    flat = watermark_samples.reshape(_ROWS, _COLS)
    out = pl.pallas_call(
        _discretize_body,
        grid=(_ROWS // _BLOCK_ROWS,),
        in_specs=[pl.BlockSpec((_BLOCK_ROWS, _COLS), lambda i: (i, 0))],
        out_specs=pl.BlockSpec((_BLOCK_ROWS, _COLS), lambda i: (i, 0)),
        out_shape=jax.ShapeDtypeStruct((_ROWS, _COLS), jnp.float32),
    )(flat)
    return (out.reshape(watermark_samples.shape), response)
